# baseline (speedup 1.0000x reference)
import functools

import jax
import jax.numpy as jnp
from jax import lax
from jax.experimental import pallas as pl
from jax.experimental.pallas import tpu as pltpu
from jax.experimental.pallas import tpu_sc as plsc

D = 32
NL = 8
CIN = 1792


def _detile_sc(Wt, Wtail):
    Dd, V = Wt.shape
    n_chunks = V // CIN
    tail = V - n_chunks * CIN
    QR = CIN // 4

    info = plsc.get_sparse_core_info()
    nw = info.num_cores * info.num_subcores
    kbase = n_chunks // nw
    kextra = n_chunks - kbase * nw

    mesh = plsc.VectorSubcoreMesh(core_axis_name="c", subcore_axis_name="s")

    @functools.partial(
        pl.kernel,
        mesh=mesh,
        out_type=jax.ShapeDtypeStruct((V * Dd // 128, 128), jnp.float32),
        scratch_types=[
            pltpu.VMEM((Dd, CIN + 1), jnp.float32),
            pltpu.VMEM((QR, 128), jnp.float32),
            pltpu.VMEM((tail, Dd), jnp.float32),
        ],
        compiler_params=pltpu.CompilerParams(
            use_tc_tiling_on_sc=True, needs_layout_passes=False),
    )
    def body(wt_hbm, wtail_hbm, out_hbm, in_v, out_v, tail_v):
        wid = lax.axis_index("s") * info.num_cores + lax.axis_index("c")
        n_k = kbase + jnp.where(wid < kextra, 1, 0)
        dvecs = [lax.iota(jnp.int32, 16), lax.iota(jnp.int32, 16) + 16]

        def do_chunk(c, ncols):
            pltpu.sync_copy(wt_hbm.at[:, pl.ds(c * CIN, ncols)],
                            in_v.at[:, pl.ds(0, ncols)])

            @plsc.parallel_loop(0, ncols // 4, unroll=4)
            def _(r):
                for h in range(8):
                    lc = jnp.full((16,), 4 * r + h // 2, jnp.int32)
                    v = plsc.load_gather(in_v, [dvecs[h % 2], lc])
                    out_v[r, pl.ds(16 * h, 16)] = v

            pltpu.sync_copy(out_v.at[pl.ds(0, ncols // 4)],
                            out_hbm.at[pl.ds(c * QR, ncols // 4)])

        def chunk_fn(k, carry):
            do_chunk(k * nw + wid, CIN)
            return carry

        lax.fori_loop(0, n_k, chunk_fn, 0)

        @pl.when(wid == nw - 1)
        def _():
            pltpu.sync_copy(wtail_hbm, tail_v)
            ng = tail * Dd // 16
            for t in range(ng):
                v = tail_v[t // 2, pl.ds(16 * (t % 2), 16)]
                out_v[t // 8, pl.ds(16 * (t % 8), 16)] = v
            pltpu.sync_copy(out_v.at[pl.ds(0, tail * Dd // 128)],
                            out_hbm.at[pl.ds(n_chunks * QR, tail * Dd // 128)])

    return body(Wt, Wtail)


def _gather_sc(xT, W, B, L):
    info = plsc.get_sparse_core_info()
    nw = info.num_cores * info.num_subcores
    assert B == nw * 128
    n_blocks = L // NL

    mesh = plsc.VectorSubcoreMesh(core_axis_name="c", subcore_axis_name="s")

    @functools.partial(
        pl.kernel,
        mesh=mesh,
        out_type=jax.ShapeDtypeStruct((L, D // 8, nw, 8, 128), jnp.float32),
        scratch_types=[
            pltpu.VMEM((2, NL, 128), jnp.int32),
            pltpu.VMEM((2, NL, 128, D), jnp.float32),
            pltpu.VMEM((NL, D, 129), jnp.float32),
            pltpu.SemaphoreType.DMA((2,)),
            pltpu.SemaphoreType.DMA,
        ],
        compiler_params=pltpu.CompilerParams(
            use_tc_tiling_on_sc=False, needs_layout_passes=False),
    )
    def body(x_hbm, w_hbm, out_hbm, idx_v, rows_v, tout_v, gsem, osem):
        wid = lax.axis_index("s") * info.num_cores + lax.axis_index("c")
        b0 = wid * 128

        halves = [lax.iota(jnp.int32, 16), lax.iota(jnp.int32, 16) + 16]

        def fire_gathers(blk, slot):
            pltpu.sync_copy(x_hbm.at[pl.ds(blk * NL, NL), pl.ds(b0, 128)],
                            idx_v.at[slot])
            for j in range(NL):
                pltpu.async_copy(w_hbm.at[idx_v.at[slot, j]],
                                 rows_v.at[slot, j], gsem.at[slot])

        def store_copies(l0):
            return [
                pltpu.make_async_copy(
                    tout_v.at[j, pl.ds(8 * td, 8), pl.ds(0, 128)],
                    out_hbm.at[l0 + j, td, wid], osem)
                for j in range(NL) for td in range(D // 8)
            ]

        fire_gathers(0, 0)

        def block(blk, carry):
            p = lax.rem(blk, 2)

            @pl.when(blk != n_blocks - 1)
            def _():
                fire_gathers(blk + 1, 1 - p)

            for j in range(NL):
                pltpu.make_async_copy(w_hbm.at[idx_v.at[p, j]],
                                      rows_v.at[p, j], gsem.at[p]).wait()

            @pl.when(blk != 0)
            def _():
                for c in store_copies((blk - 1) * NL):
                    c.wait()

            l0 = blk * NL
            for j in range(NL):
                src = rows_v.at[p, j]
                dst = tout_v.at[j]

                @plsc.parallel_loop(0, 128, unroll=8)
                def _(i):
                    isplat = jnp.full((16,), i, jnp.int32)
                    for h in range(2):
                        v = src[i, pl.ds(16 * h, 16)]
                        plsc.store_scatter(dst, [halves[h], isplat], v)
            for c in store_copies(l0):
                c.start()
            return carry

        lax.fori_loop(0, n_blocks, block, 0)

        for c in store_copies((n_blocks - 1) * NL):
            c.wait()

    return body(xT, W)


def kernel(x, W):
    B, L = x.shape
    Vfull = (W.shape[0] // 128) * 128
    Wlin = _detile_sc(jnp.transpose(W), W[Vfull:]).reshape(W.shape[0], D)
    xT = jnp.transpose(x.astype(jnp.int32))
    out5 = _gather_sc(xT, Wlin, B, L)
    return out5.transpose(2, 4, 0, 1, 3).reshape(B, L, D)


# speedup vs baseline: 1.0679x; 1.0679x over previous
import functools

import jax
import jax.numpy as jnp
from jax import lax
from jax.experimental import pallas as pl
from jax.experimental.pallas import tpu as pltpu
from jax.experimental.pallas import tpu_sc as plsc

D = 32
NL = 8


def _gather_sc(xT, W, B, L):
    info = plsc.get_sparse_core_info()
    nw = info.num_cores * info.num_subcores
    assert B == nw * 128
    n_blocks = L // NL

    mesh = plsc.VectorSubcoreMesh(core_axis_name="c", subcore_axis_name="s")

    @functools.partial(
        pl.kernel,
        mesh=mesh,
        out_type=jax.ShapeDtypeStruct((L, D // 8, nw, 8, 128), jnp.float32),
        scratch_types=[
            pltpu.VMEM((2, NL, 128), jnp.int32),
            pltpu.VMEM((2, NL, 128, D), jnp.float32),
            pltpu.VMEM((NL, D, 129), jnp.float32),
            pltpu.SemaphoreType.DMA((2,)),
            pltpu.SemaphoreType.DMA,
        ],
        compiler_params=pltpu.CompilerParams(
            use_tc_tiling_on_sc=False, needs_layout_passes=False),
    )
    def body(x_hbm, w_hbm, out_hbm, idx_v, rows_v, tout_v, gsem, osem):
        wid = lax.axis_index("s") * info.num_cores + lax.axis_index("c")
        b0 = wid * 128

        halves = [lax.iota(jnp.int32, 16), lax.iota(jnp.int32, 16) + 16]

        def fire_gathers(blk, slot):
            pltpu.sync_copy(x_hbm.at[pl.ds(blk * NL, NL), pl.ds(b0, 128)],
                            idx_v.at[slot])
            for j in range(NL):
                pltpu.async_copy(w_hbm.at[idx_v.at[slot, j]],
                                 rows_v.at[slot, j], gsem.at[slot])

        def store_copies(l0):
            return [
                pltpu.make_async_copy(
                    tout_v.at[j, pl.ds(8 * td, 8), pl.ds(0, 128)],
                    out_hbm.at[l0 + j, td, wid], osem)
                for j in range(NL) for td in range(D // 8)
            ]

        fire_gathers(0, 0)

        def block(blk, carry):
            p = lax.rem(blk, 2)

            @pl.when(blk != n_blocks - 1)
            def _():
                fire_gathers(blk + 1, 1 - p)

            for j in range(NL):
                pltpu.make_async_copy(w_hbm.at[idx_v.at[p, j]],
                                      rows_v.at[p, j], gsem.at[p]).wait()

            @pl.when(blk != 0)
            def _():
                for c in store_copies((blk - 1) * NL):
                    c.wait()

            l0 = blk * NL
            for j in range(NL):
                src = rows_v.at[p, j]
                dst = tout_v.at[j]

                @plsc.parallel_loop(0, 128, unroll=8)
                def _(i):
                    isplat = jnp.full((16,), i, jnp.int32)
                    for h in range(2):
                        v = src[i, pl.ds(16 * h, 16)]
                        plsc.store_scatter(dst, [halves[h], isplat], v)
            for c in store_copies(l0):
                c.start()
            return carry

        lax.fori_loop(0, n_blocks, block, 0)

        for c in store_copies((n_blocks - 1) * NL):
            c.wait()

    return body(xT, W)


def kernel(x, W):
    B, L = x.shape
    xT = jnp.transpose(x.astype(jnp.int32) * 4)
    W4 = jnp.pad(W, ((0, 0), (0, 96))).reshape(4 * W.shape[0], D)
    out5 = _gather_sc(xT, W4, B, L)
    return out5.transpose(2, 4, 0, 1, 3).reshape(B, L, D)


# speedup vs baseline: 1.2031x; 1.1266x over previous
import functools

import jax
import jax.numpy as jnp
from jax import lax
from jax.experimental import pallas as pl
from jax.experimental.pallas import tpu as pltpu
from jax.experimental.pallas import tpu_sc as plsc

D = 32
NL = 8
CIN = 896


def _detile_sc(Wt, Wtail):
    Dd, V = Wt.shape
    n_chunks = V // CIN
    tail = V - n_chunks * CIN
    QR = CIN * Dd // 128

    info = plsc.get_sparse_core_info()
    nw = info.num_cores * info.num_subcores
    kbase = n_chunks // nw
    kextra = n_chunks - kbase * nw

    mesh = plsc.VectorSubcoreMesh(core_axis_name="c", subcore_axis_name="s")

    @functools.partial(
        pl.kernel,
        mesh=mesh,
        out_type=jax.ShapeDtypeStruct((V * Dd // 128, 128), jnp.float32),
        scratch_types=[
            pltpu.VMEM((2, Dd, CIN + 1), jnp.float32),
            pltpu.VMEM((2, QR, 128), jnp.float32),
            pltpu.VMEM((tail, Dd), jnp.float32),
            pltpu.SemaphoreType.DMA((2,)),
            pltpu.SemaphoreType.DMA,
        ],
        compiler_params=pltpu.CompilerParams(
            use_tc_tiling_on_sc=True, needs_layout_passes=False),
    )
    def body(wt_hbm, wtail_hbm, out_hbm, in_v, out_v, tail_v, isem, osem):
        wid = lax.axis_index("s") * info.num_cores + lax.axis_index("c")
        n_k = kbase + jnp.where(wid < kextra, 1, 0)
        dvecs = [lax.iota(jnp.int32, 16), lax.iota(jnp.int32, 16) + 16]

        def in_copy(c, slot):
            return pltpu.make_async_copy(
                wt_hbm.at[:, pl.ds(c * CIN, CIN)],
                in_v.at[slot, :, pl.ds(0, CIN)], isem.at[slot])

        def out_copy(c, slot):
            return pltpu.make_async_copy(
                out_v.at[slot], out_hbm.at[pl.ds(c * QR, QR)], osem)

        in_copy(wid, 0).start()

        def chunk_fn(k, carry):
            p = lax.rem(k, 2)
            c = k * nw + wid

            @pl.when(k != n_k - 1)
            def _():
                in_copy(c + nw, 1 - p).start()

            in_copy(c, p).wait()

            @pl.when(k >= 2)
            def _():
                out_copy(c, p).wait()

            @plsc.parallel_loop(0, QR, unroll=4)
            def _(r):
                for h in range(8):
                    lc = jnp.full((16,), 4 * r + h // 2, jnp.int32)
                    v = plsc.load_gather(in_v.at[p], [dvecs[h % 2], lc])
                    out_v[p, r, pl.ds(16 * h, 16)] = v

            out_copy(c, p).start()
            return carry

        lax.fori_loop(0, n_k, chunk_fn, 0)
        out_copy(0, 0).wait()
        out_copy(0, 1).wait()

        @pl.when(wid == nw - 1)
        def _():
            pltpu.sync_copy(wtail_hbm, tail_v)
            for t in range(tail * Dd // 16):
                v = tail_v[t // 2, pl.ds(16 * (t % 2), 16)]
                out_v[0, t // 8, pl.ds(16 * (t % 8), 16)] = v
            pltpu.sync_copy(out_v.at[0, pl.ds(0, tail * Dd // 128)],
                            out_hbm.at[pl.ds(n_chunks * QR, tail * Dd // 128)])

    return body(Wt, Wtail)


def _gather_sc(xT, W, B, L):
    info = plsc.get_sparse_core_info()
    nw = info.num_cores * info.num_subcores
    assert B == nw * 128
    n_blocks = L // NL

    mesh = plsc.VectorSubcoreMesh(core_axis_name="c", subcore_axis_name="s")

    @functools.partial(
        pl.kernel,
        mesh=mesh,
        out_type=jax.ShapeDtypeStruct((L, D // 8, nw, 8, 128), jnp.float32),
        scratch_types=[
            pltpu.VMEM((2, NL, 128), jnp.int32),
            pltpu.VMEM((2, NL, 128, D), jnp.float32),
            pltpu.VMEM((NL, D, 129), jnp.float32),
            pltpu.SemaphoreType.DMA((2,)),
            pltpu.SemaphoreType.DMA,
        ],
        compiler_params=pltpu.CompilerParams(
            use_tc_tiling_on_sc=False, needs_layout_passes=False),
    )
    def body(x_hbm, w_hbm, out_hbm, idx_v, rows_v, tout_v, gsem, osem):
        wid = lax.axis_index("s") * info.num_cores + lax.axis_index("c")
        b0 = wid * 128

        halves = [lax.iota(jnp.int32, 16), lax.iota(jnp.int32, 16) + 16]

        def fire_gathers(blk, slot):
            pltpu.sync_copy(x_hbm.at[pl.ds(blk * NL, NL), pl.ds(b0, 128)],
                            idx_v.at[slot])
            for j in range(NL):
                pltpu.async_copy(w_hbm.at[idx_v.at[slot, j]],
                                 rows_v.at[slot, j], gsem.at[slot])

        def store_copies(l0):
            return [
                pltpu.make_async_copy(
                    tout_v.at[j, pl.ds(8 * td, 8), pl.ds(0, 128)],
                    out_hbm.at[l0 + j, td, wid], osem)
                for j in range(NL) for td in range(D // 8)
            ]

        fire_gathers(0, 0)

        def block(blk, carry):
            p = lax.rem(blk, 2)

            @pl.when(blk != n_blocks - 1)
            def _():
                fire_gathers(blk + 1, 1 - p)

            for j in range(NL):
                pltpu.make_async_copy(w_hbm.at[idx_v.at[p, j]],
                                      rows_v.at[p, j], gsem.at[p]).wait()

            @pl.when(blk != 0)
            def _():
                for c in store_copies((blk - 1) * NL):
                    c.wait()

            l0 = blk * NL
            for j in range(NL):
                src = rows_v.at[p, j]
                dst = tout_v.at[j]

                @plsc.parallel_loop(0, 128, unroll=8)
                def _(i):
                    isplat = jnp.full((16,), i, jnp.int32)
                    for h in range(2):
                        v = src[i, pl.ds(16 * h, 16)]
                        plsc.store_scatter(dst, [halves[h], isplat], v)
            for c in store_copies(l0):
                c.start()
            return carry

        lax.fori_loop(0, n_blocks, block, 0)

        for c in store_copies((n_blocks - 1) * NL):
            c.wait()

    return body(xT, W)


def kernel(x, W):
    B, L = x.shape
    xT = jnp.transpose(x.astype(jnp.int32))
    Vfull = (W.shape[0] // 128) * 128
    Wlin = _detile_sc(jnp.transpose(W), W[Vfull:]).reshape(W.shape[0], D)
    out5 = _gather_sc(xT, Wlin, B, L)
    return out5.transpose(2, 4, 0, 1, 3).reshape(B, L, D)
